# baseline (speedup 1.0000x reference)
import functools

import jax
import jax.numpy as jnp
from jax import lax
from jax.experimental import pallas as pl
from jax.experimental.pallas import tpu as pltpu
from jax.experimental.pallas import tpu_sc as plsc

N = 10000
E = 320000
D = 128
G = 64

NUM_CORES = 2
NUM_SUBCORES = 16
CHUNK = 128
A_CH = 104
B_CH = 56
CORE1_CHUNKS = NUM_SUBCORES * B_CH
TOT_CH = CORE1_CHUNKS + NUM_SUBCORES * A_CH
EPAD = TOT_CH * CHUNK
ZBLK = 128
ROWS_PER_SUB = 640
NPAD = NUM_SUBCORES * ROWS_PER_SUB

_mesh = plsc.VectorSubcoreMesh(core_axis_name="c", subcore_axis_name="s")


@functools.partial(
    pl.kernel,
    out_type=(jax.ShapeDtypeStruct((NPAD, D), jnp.float32),
              jax.ShapeDtypeStruct((NPAD, D), jnp.float32)),
    mesh=_mesh,
    scratch_types=[
        pltpu.VMEM((A_CH, CHUNK), jnp.int32),
        pltpu.VMEM((A_CH, CHUNK), jnp.int32),
        pltpu.VMEM((CHUNK, D), jnp.float32),
        pltpu.VMEM_SHARED((NPAD, D), jnp.float32),
    ],
)
def _sc_agg_kernel(h_hbm, src_hbm, dst_hbm, z_hbm, p0_hbm, p1_hbm,
                   src_v, dst_v, rows_v, acc_sh):
    cid = lax.axis_index("c")
    sid = lax.axis_index("s")
    base = sid * ROWS_PER_SUB

    off = 0
    while off < ROWS_PER_SUB:
        zn = min(ZBLK, ROWS_PER_SUB - off)
        pltpu.sync_copy(z_hbm.at[pl.ds(0, zn)], acc_sh.at[pl.ds(base + off, zn)])
        off += zn
    plsc.subcore_barrier()

    start = jnp.where(cid == 0, CORE1_CHUNKS + sid * A_CH, sid * B_CH)
    nch = jnp.where(cid == 0, A_CH, B_CH)
    pltpu.sync_copy(src_hbm.at[pl.ds(start, A_CH)], src_v)
    pltpu.sync_copy(dst_hbm.at[pl.ds(start, A_CH)], dst_v)

    @pl.loop(0, nch)
    def _(j):
        pltpu.sync_copy(h_hbm.at[src_v.at[j]], rows_v)
        pltpu.sync_copy(rows_v, acc_sh.at[dst_v.at[j]], add=True)

    plsc.subcore_barrier()

    sl = pl.ds(base, ROWS_PER_SUB)

    @pl.when(cid == 0)
    def _():
        pltpu.sync_copy(acc_sh.at[sl], p0_hbm.at[sl])

    @pl.when(cid == 1)
    def _():
        pltpu.sync_copy(acc_sh.at[sl], p1_hbm.at[sl])


BR = 1000
NBLK = N // BR


def _mlp_body(x_ref, p0_ref, p1_ref, wa_ref, ba_ref, wb_ref, bb_ref, o_ref):
    h = x_ref[...] + p0_ref[...] + p1_ref[...]
    a = jnp.maximum(
        jnp.dot(h, wa_ref[...], preferred_element_type=jnp.float32) + ba_ref[...],
        0.0)
    o_ref[...] = jnp.dot(a, wb_ref[...], preferred_element_type=jnp.float32) + bb_ref[...]


def _tc_mlp(x, p0, p1, Wa, ba, Wb, bb):
    return pl.pallas_call(
        _mlp_body,
        grid=(NBLK,),
        in_specs=[
            pl.BlockSpec((BR, D), lambda i: (i, 0)),
            pl.BlockSpec((BR, D), lambda i: (i, 0)),
            pl.BlockSpec((BR, D), lambda i: (i, 0)),
            pl.BlockSpec((D, D), lambda i: (0, 0)),
            pl.BlockSpec((1, D), lambda i: (0, 0)),
            pl.BlockSpec((D, D), lambda i: (0, 0)),
            pl.BlockSpec((1, D), lambda i: (0, 0)),
        ],
        out_specs=pl.BlockSpec((BR, D), lambda i: (i, 0)),
        out_shape=jax.ShapeDtypeStruct((N, D), jnp.float32),
    )(x, p0, p1, Wa, ba.reshape(1, D), Wb, bb.reshape(1, D))


def _final_body(h_ref, p0_ref, p1_ref, b_ref, wa_ref, ba_ref, wb_ref, bb_ref,
                wl_ref, bl_ref, o_ref, acc_s, acc_c):
    i = pl.program_id(0)
    h = h_ref[...] + p0_ref[...] + p1_ref[...]
    a = jnp.maximum(
        jnp.dot(h, wa_ref[...], preferred_element_type=jnp.float32) + ba_ref[...],
        0.0)
    h2 = jnp.dot(a, wb_ref[...], preferred_element_type=jnp.float32) + bb_ref[...]

    seg = lax.broadcasted_iota(jnp.int32, (BR, G), 1)
    onehot = (b_ref[...] == seg).astype(jnp.float32)
    contrib = lax.dot_general(onehot, h2, (((0,), (0,)), ((), ())),
                              preferred_element_type=jnp.float32)
    cnt = lax.dot_general(onehot, jnp.ones((BR, 1), jnp.float32),
                          (((0,), (0,)), ((), ())),
                          preferred_element_type=jnp.float32)

    @pl.when(i == 0)
    def _():
        acc_s[...] = jnp.zeros_like(acc_s)
        acc_c[...] = jnp.zeros_like(acc_c)

    acc_s[...] += contrib
    acc_c[...] += cnt

    @pl.when(i == NBLK - 1)
    def _():
        pooled = acc_s[...] / jnp.maximum(acc_c[...], 1.0)
        o_ref[...] = (jnp.dot(pooled, wl_ref[...],
                              preferred_element_type=jnp.float32) + bl_ref[...])


def _tc_final(h1, p0, p1, bcol, Wa, ba, Wb, bb, Wl, bl):
    return pl.pallas_call(
        _final_body,
        grid=(NBLK,),
        in_specs=[
            pl.BlockSpec((BR, D), lambda i: (i, 0)),
            pl.BlockSpec((BR, D), lambda i: (i, 0)),
            pl.BlockSpec((BR, D), lambda i: (i, 0)),
            pl.BlockSpec((BR, 1), lambda i: (i, 0)),
            pl.BlockSpec((D, D), lambda i: (0, 0)),
            pl.BlockSpec((1, D), lambda i: (0, 0)),
            pl.BlockSpec((D, D), lambda i: (0, 0)),
            pl.BlockSpec((1, D), lambda i: (0, 0)),
            pl.BlockSpec((D, D), lambda i: (0, 0)),
            pl.BlockSpec((1, D), lambda i: (0, 0)),
        ],
        out_specs=pl.BlockSpec((G, D), lambda i: (0, 0)),
        out_shape=jax.ShapeDtypeStruct((G, D), jnp.float32),
        scratch_shapes=[
            pltpu.VMEM((G, D), jnp.float32),
            pltpu.VMEM((G, 1), jnp.float32),
        ],
    )(h1, p0, p1, bcol, Wa, ba.reshape(1, D), Wb, bb.reshape(1, D),
      Wl, bl.reshape(1, D))


def kernel(x, edge_index, batch, W1a, b1a, W1b, b1b, W2a, b2a, W2b, b2b, Wl, bl):
    src = edge_index[0]
    dst = edge_index[1]
    pad = EPAD - E
    srcr = jnp.concatenate([src, jnp.zeros((pad,), jnp.int32)]).reshape(TOT_CH, CHUNK)
    dstr = jnp.concatenate([dst, jnp.full((pad,), N, jnp.int32)]).reshape(TOT_CH, CHUNK)
    zblk = jnp.zeros((ZBLK, D), jnp.float32)
    bcol = batch.reshape(N, 1)

    p0, p1 = _sc_agg_kernel(x, srcr, dstr, zblk)
    h1 = _tc_mlp(x, p0, p1, W1a, b1a, W1b, b1b)
    q0, q1 = _sc_agg_kernel(h1, srcr, dstr, zblk)
    return _tc_final(h1, q0, q1, bcol, W2a, b2a, W2b, b2b, Wl, bl)


# speedup vs baseline: 1.6829x; 1.6829x over previous
import functools

import jax
import jax.numpy as jnp
from jax import lax
from jax.experimental import pallas as pl
from jax.experimental.pallas import tpu as pltpu
from jax.experimental.pallas import tpu_sc as plsc

N = 10000
E = 320000
D = 128
G = 64

NUM_CORES = 2
NUM_SUBCORES = 16
NW = NUM_CORES * NUM_SUBCORES
CHUNK = 128
NCH = -(-E // (NW * CHUNK))
EPAD = NW * NCH * CHUNK
ROWS_PER_SUB = 640
NPAD = NUM_SUBCORES * ROWS_PER_SUB

_mesh = plsc.VectorSubcoreMesh(core_axis_name="c", subcore_axis_name="s")


@functools.partial(
    pl.kernel,
    out_type=(jax.ShapeDtypeStruct((NPAD, D), jnp.float32),
              jax.ShapeDtypeStruct((NPAD, D), jnp.float32)),
    mesh=_mesh,
    scratch_types=[
        pltpu.VMEM((NCH, CHUNK), jnp.int32),
        pltpu.VMEM((NCH, CHUNK), jnp.int32),
        pltpu.VMEM((CHUNK, D), jnp.float32),
        pltpu.VMEM_SHARED((NPAD, D), jnp.float32),
    ],
)
def _sc_agg_kernel(h_hbm, src_hbm, dst_hbm, z_hbm, p0_hbm, p1_hbm,
                   src_v, dst_v, rows_v, acc_sh):
    cid = lax.axis_index("c")
    sid = lax.axis_index("s")
    wid = sid * NUM_CORES + cid
    base = sid * ROWS_PER_SUB

    for k in range(ROWS_PER_SUB // CHUNK):
        pltpu.sync_copy(z_hbm, acc_sh.at[pl.ds(base + k * CHUNK, CHUNK)])
    plsc.subcore_barrier()

    pltpu.sync_copy(src_hbm.at[wid], src_v)
    pltpu.sync_copy(dst_hbm.at[wid], dst_v)

    @pl.loop(0, NCH)
    def _(j):
        pltpu.sync_copy(h_hbm.at[src_v.at[j]], rows_v)
        pltpu.sync_copy(rows_v, acc_sh.at[dst_v.at[j]], add=True)

    plsc.subcore_barrier()

    sl = pl.ds(base, ROWS_PER_SUB)

    @pl.when(cid == 0)
    def _():
        pltpu.sync_copy(acc_sh.at[sl], p0_hbm.at[sl])

    @pl.when(cid == 1)
    def _():
        pltpu.sync_copy(acc_sh.at[sl], p1_hbm.at[sl])


BR = 1000
NBLK = N // BR


def _mlp_body(x_ref, p0_ref, p1_ref, wa_ref, ba_ref, wb_ref, bb_ref, o_ref):
    h = x_ref[...] + p0_ref[...] + p1_ref[...]
    a = jnp.maximum(
        jnp.dot(h, wa_ref[...], preferred_element_type=jnp.float32) + ba_ref[...],
        0.0)
    o_ref[...] = jnp.dot(a, wb_ref[...], preferred_element_type=jnp.float32) + bb_ref[...]


def _tc_mlp(x, p0, p1, Wa, ba, Wb, bb):
    return pl.pallas_call(
        _mlp_body,
        grid=(NBLK,),
        in_specs=[
            pl.BlockSpec((BR, D), lambda i: (i, 0)),
            pl.BlockSpec((BR, D), lambda i: (i, 0)),
            pl.BlockSpec((BR, D), lambda i: (i, 0)),
            pl.BlockSpec((D, D), lambda i: (0, 0)),
            pl.BlockSpec((1, D), lambda i: (0, 0)),
            pl.BlockSpec((D, D), lambda i: (0, 0)),
            pl.BlockSpec((1, D), lambda i: (0, 0)),
        ],
        out_specs=pl.BlockSpec((BR, D), lambda i: (i, 0)),
        out_shape=jax.ShapeDtypeStruct((N, D), jnp.float32),
    )(x, p0, p1, Wa, ba.reshape(1, D), Wb, bb.reshape(1, D))


def _final_body(h_ref, p0_ref, p1_ref, b_ref, wa_ref, ba_ref, wb_ref, bb_ref,
                wl_ref, bl_ref, o_ref, acc_s, acc_c):
    i = pl.program_id(0)
    h = h_ref[...] + p0_ref[...] + p1_ref[...]
    a = jnp.maximum(
        jnp.dot(h, wa_ref[...], preferred_element_type=jnp.float32) + ba_ref[...],
        0.0)
    h2 = jnp.dot(a, wb_ref[...], preferred_element_type=jnp.float32) + bb_ref[...]

    seg = lax.broadcasted_iota(jnp.int32, (BR, G), 1)
    onehot = (b_ref[...] == seg).astype(jnp.float32)
    contrib = lax.dot_general(onehot, h2, (((0,), (0,)), ((), ())),
                              preferred_element_type=jnp.float32)
    cnt = lax.dot_general(onehot, jnp.ones((BR, 1), jnp.float32),
                          (((0,), (0,)), ((), ())),
                          preferred_element_type=jnp.float32)

    @pl.when(i == 0)
    def _():
        acc_s[...] = jnp.zeros_like(acc_s)
        acc_c[...] = jnp.zeros_like(acc_c)

    acc_s[...] += contrib
    acc_c[...] += cnt

    @pl.when(i == NBLK - 1)
    def _():
        pooled = acc_s[...] / jnp.maximum(acc_c[...], 1.0)
        o_ref[...] = (jnp.dot(pooled, wl_ref[...],
                              preferred_element_type=jnp.float32) + bl_ref[...])


def _tc_final(h1, p0, p1, bcol, Wa, ba, Wb, bb, Wl, bl):
    return pl.pallas_call(
        _final_body,
        grid=(NBLK,),
        in_specs=[
            pl.BlockSpec((BR, D), lambda i: (i, 0)),
            pl.BlockSpec((BR, D), lambda i: (i, 0)),
            pl.BlockSpec((BR, D), lambda i: (i, 0)),
            pl.BlockSpec((BR, 1), lambda i: (i, 0)),
            pl.BlockSpec((D, D), lambda i: (0, 0)),
            pl.BlockSpec((1, D), lambda i: (0, 0)),
            pl.BlockSpec((D, D), lambda i: (0, 0)),
            pl.BlockSpec((1, D), lambda i: (0, 0)),
            pl.BlockSpec((D, D), lambda i: (0, 0)),
            pl.BlockSpec((1, D), lambda i: (0, 0)),
        ],
        out_specs=pl.BlockSpec((G, D), lambda i: (0, 0)),
        out_shape=jax.ShapeDtypeStruct((G, D), jnp.float32),
        scratch_shapes=[
            pltpu.VMEM((G, D), jnp.float32),
            pltpu.VMEM((G, 1), jnp.float32),
        ],
    )(h1, p0, p1, bcol, Wa, ba.reshape(1, D), Wb, bb.reshape(1, D),
      Wl, bl.reshape(1, D))


def kernel(x, edge_index, batch, W1a, b1a, W1b, b1b, W2a, b2a, W2b, b2b, Wl, bl):
    src = edge_index[0]
    dst = edge_index[1]
    pad = EPAD - E
    srcr = jnp.concatenate([src, jnp.zeros((pad,), jnp.int32)]).reshape(NW, NCH, CHUNK)
    dstr = jnp.concatenate([dst, jnp.full((pad,), N, jnp.int32)]).reshape(NW, NCH, CHUNK)
    zblk = jnp.zeros((CHUNK, D), jnp.float32)
    bcol = batch.reshape(N, 1)

    p0, p1 = _sc_agg_kernel(x, srcr, dstr, zblk)
    h1 = _tc_mlp(x, p0, p1, W1a, b1a, W1b, b1b)
    q0, q1 = _sc_agg_kernel(h1, srcr, dstr, zblk)
    return _tc_final(h1, q0, q1, bcol, W2a, b2a, W2b, b2b, Wl, bl)
